# Initial kernel scaffold; baseline (speedup 1.0000x reference)
import functools

import jax
import jax.numpy as jnp
from jax import lax
from jax.experimental import pallas as pl
from jax.experimental.pallas import tpu as pltpu
from jax.experimental.pallas import tpu_sc as plsc

N_NODES = 10000
N_EDGES = 320000
F_IN = 128
HID = 128
C_OUT = 64

NC = 2
NS = 16
NW = NC * NS

CHUNK = 128
EDGES_PER_TILE = 10240
CH_PER_TILE = EDGES_PER_TILE // CHUNK
E_PAD = EDGES_PER_TILE * NW
N_ACC = 10240
ROWS_OUT = N_ACC // NS
W1 = 144
W2 = 64

BLK_M = 1024
GRID_M = 10


def _make_sc_pass(width):
    mesh = plsc.VectorSubcoreMesh(core_axis_name="c", subcore_axis_name="s")

    @functools.partial(
        pl.kernel,
        out_type=jax.ShapeDtypeStruct((NC, N_ACC, width), jnp.float32),
        mesh=mesh,
        scratch_types=[
            pltpu.VMEM((CH_PER_TILE, CHUNK), jnp.int32),
            pltpu.VMEM((CH_PER_TILE, CHUNK), jnp.int32),
            pltpu.VMEM((CHUNK, width), jnp.float32),
            pltpu.VMEM_SHARED((N_ACC, width), jnp.float32),
        ],
    )
    def sc_pass(table, src_idx, dst_idx, zeros, out, src_v, dst_v, rows_v, acc):
        c = lax.axis_index("c")
        s = lax.axis_index("s")
        wid = c * NS + s
        pltpu.sync_copy(zeros, acc.at[pl.ds(s * ROWS_OUT, ROWS_OUT)])
        pltpu.sync_copy(src_idx.at[wid], src_v)
        pltpu.sync_copy(dst_idx.at[wid], dst_v)
        plsc.subcore_barrier()

        @pl.loop(0, CH_PER_TILE)
        def _(j):
            pltpu.sync_copy(table.at[src_v.at[j]], rows_v)
            pltpu.sync_copy(rows_v, acc.at[dst_v.at[j]], add=True)

        plsc.subcore_barrier()
        pltpu.sync_copy(
            acc.at[pl.ds(s * ROWS_OUT, ROWS_OUT)],
            out.at[c, pl.ds(s * ROWS_OUT, ROWS_OUT)],
        )

    return sc_pass


_sc_pass1 = _make_sc_pass(W1)
_sc_pass2 = _make_sc_pass(W2)


def _tc_mid_body(x_ref, parts_ref, w1s_ref, w1n_ref, b1_ref, w2n_ref, w2s_ref,
                 b2_ref, hn_ref, hs_ref, dinv_ref):
    p = parts_ref[0, :, :F_IN] + parts_ref[1, :, :F_IN]
    deg = parts_ref[0, :, F_IN:F_IN + 1] + parts_ref[1, :, F_IN:F_IN + 1]
    dinv = 1.0 / jnp.maximum(deg, 1.0)
    mean = p * dinv
    h = x_ref[...] @ w1s_ref[...] + mean @ w1n_ref[...] + b1_ref[...]
    h = jnp.maximum(h, 0.0)
    hn_ref[...] = h @ w2n_ref[...]
    hs_ref[...] = h @ w2s_ref[...] + b2_ref[...]
    dinv_ref[...] = dinv


def _tc_fin_body(hs_ref, parts_ref, dinv_ref, out_ref):
    agg = parts_ref[0] + parts_ref[1]
    out_ref[...] = hs_ref[...] + agg * dinv_ref[...]


_tc_mid = pl.pallas_call(
    _tc_mid_body,
    grid=(GRID_M,),
    in_specs=[
        pl.BlockSpec((BLK_M, F_IN), lambda i: (i, 0)),
        pl.BlockSpec((NC, BLK_M, W1), lambda i: (0, i, 0)),
        pl.BlockSpec((F_IN, HID), lambda i: (0, 0)),
        pl.BlockSpec((F_IN, HID), lambda i: (0, 0)),
        pl.BlockSpec((1, HID), lambda i: (0, 0)),
        pl.BlockSpec((HID, C_OUT), lambda i: (0, 0)),
        pl.BlockSpec((HID, C_OUT), lambda i: (0, 0)),
        pl.BlockSpec((1, C_OUT), lambda i: (0, 0)),
    ],
    out_specs=[
        pl.BlockSpec((BLK_M, W2), lambda i: (i, 0)),
        pl.BlockSpec((BLK_M, C_OUT), lambda i: (i, 0)),
        pl.BlockSpec((BLK_M, 1), lambda i: (i, 0)),
    ],
    out_shape=[
        jax.ShapeDtypeStruct((N_NODES, W2), jnp.float32),
        jax.ShapeDtypeStruct((N_NODES, C_OUT), jnp.float32),
        jax.ShapeDtypeStruct((N_NODES, 1), jnp.float32),
    ],
)

_tc_fin = pl.pallas_call(
    _tc_fin_body,
    grid=(GRID_M,),
    in_specs=[
        pl.BlockSpec((BLK_M, C_OUT), lambda i: (i, 0)),
        pl.BlockSpec((NC, BLK_M, C_OUT), lambda i: (0, i, 0)),
        pl.BlockSpec((BLK_M, 1), lambda i: (i, 0)),
    ],
    out_specs=pl.BlockSpec((BLK_M, C_OUT), lambda i: (i, 0)),
    out_shape=jax.ShapeDtypeStruct((N_NODES, C_OUT), jnp.float32),
)


@jax.jit
def kernel(x, adj, W1_self, W1_neigh, b1, W2_self, W2_neigh, b2):
    src = adj[0]
    dst = adj[1]
    pad = E_PAD - N_EDGES
    src_p = jnp.concatenate([src, jnp.zeros((pad,), jnp.int32)])
    dst_p = jnp.concatenate([dst, jnp.full((pad,), N_NODES, jnp.int32)])
    src_p = src_p.reshape(NW, CH_PER_TILE, CHUNK)
    dst_p = dst_p.reshape(NW, CH_PER_TILE, CHUNK)

    table1 = jnp.concatenate(
        [x, jnp.ones((N_NODES, 1), jnp.float32),
         jnp.zeros((N_NODES, W1 - F_IN - 1), jnp.float32)], axis=1)

    z1 = jnp.zeros((ROWS_OUT, W1), jnp.float32)
    z2 = jnp.zeros((ROWS_OUT, W2), jnp.float32)

    parts1 = _sc_pass1(table1, src_p, dst_p, z1)
    hn, hs, dinv = _tc_mid(x, parts1, W1_self, W1_neigh, b1.reshape(1, HID),
                           W2_neigh, W2_self, b2.reshape(1, C_OUT))
    parts2 = _sc_pass2(hn, src_p, dst_p, z2)
    return _tc_fin(hs, parts2, dinv)


# speedup vs baseline: 4.0510x; 4.0510x over previous
import functools

import jax
import jax.numpy as jnp
from jax import lax
from jax.experimental import pallas as pl
from jax.experimental.pallas import tpu as pltpu
from jax.experimental.pallas import tpu_sc as plsc

N_NODES = 10000
N_EDGES = 320000
F_IN = 128
HID = 128
C_OUT = 64

NC = 2
NS = 16
NW = NC * NS

CHUNK = 128
EDGES_PER_TILE = 10240
CH_PER_TILE = EDGES_PER_TILE // CHUNK
E_PAD = EDGES_PER_TILE * NW
N_ACC = 10240
ROWS_OUT = N_ACC // NS
W1 = 144
W2 = 64

BLK_M = 1024
GRID_M = 10


def _make_sc_pass(width):
    mesh = plsc.VectorSubcoreMesh(core_axis_name="c", subcore_axis_name="s")

    @functools.partial(
        pl.kernel,
        out_type=jax.ShapeDtypeStruct((NC, N_ACC, width), jnp.float32),
        mesh=mesh,
        compiler_params=pltpu.CompilerParams(use_tc_tiling_on_sc=False),
        scratch_types=[
            pltpu.VMEM((CH_PER_TILE, CHUNK), jnp.int32),
            pltpu.VMEM((CH_PER_TILE, CHUNK), jnp.int32),
            pltpu.VMEM((CHUNK, width), jnp.float32),
            pltpu.VMEM_SHARED((N_ACC, width), jnp.float32),
        ],
    )
    def sc_pass(table, src_idx, dst_idx, zeros, out, src_v, dst_v, rows_v, acc):
        c = lax.axis_index("c")
        s = lax.axis_index("s")
        wid = c * NS + s
        pltpu.sync_copy(zeros, acc.at[pl.ds(s * ROWS_OUT, ROWS_OUT)])
        pltpu.sync_copy(src_idx.at[wid], src_v)
        pltpu.sync_copy(dst_idx.at[wid], dst_v)
        plsc.subcore_barrier()

        @pl.loop(0, CH_PER_TILE)
        def _(j):
            pltpu.sync_copy(table.at[src_v.at[j]], rows_v)
            pltpu.sync_copy(rows_v, acc.at[dst_v.at[j]], add=True)

        plsc.subcore_barrier()
        pltpu.sync_copy(
            acc.at[pl.ds(s * ROWS_OUT, ROWS_OUT)],
            out.at[c, pl.ds(s * ROWS_OUT, ROWS_OUT)],
        )

    return sc_pass


_sc_pass1 = _make_sc_pass(W1)
_sc_pass2 = _make_sc_pass(W2)


def _tc_mid_body(x_ref, parts_ref, w1s_ref, w1n_ref, b1_ref, w2n_ref, w2s_ref,
                 b2_ref, hn_ref, hs_ref, dinv_ref):
    p = parts_ref[0, :, :F_IN] + parts_ref[1, :, :F_IN]
    deg = parts_ref[0, :, F_IN:F_IN + 1] + parts_ref[1, :, F_IN:F_IN + 1]
    dinv = 1.0 / jnp.maximum(deg, 1.0)
    mean = p * dinv
    h = x_ref[...] @ w1s_ref[...] + mean @ w1n_ref[...] + b1_ref[...]
    h = jnp.maximum(h, 0.0)
    hn_ref[...] = h @ w2n_ref[...]
    hs_ref[...] = h @ w2s_ref[...] + b2_ref[...]
    dinv_ref[...] = dinv


def _tc_fin_body(hs_ref, parts_ref, dinv_ref, out_ref):
    agg = parts_ref[0] + parts_ref[1]
    out_ref[...] = hs_ref[...] + agg * dinv_ref[...]


_tc_mid = pl.pallas_call(
    _tc_mid_body,
    grid=(GRID_M,),
    in_specs=[
        pl.BlockSpec((BLK_M, F_IN), lambda i: (i, 0)),
        pl.BlockSpec((NC, BLK_M, W1), lambda i: (0, i, 0)),
        pl.BlockSpec((F_IN, HID), lambda i: (0, 0)),
        pl.BlockSpec((F_IN, HID), lambda i: (0, 0)),
        pl.BlockSpec((1, HID), lambda i: (0, 0)),
        pl.BlockSpec((HID, C_OUT), lambda i: (0, 0)),
        pl.BlockSpec((HID, C_OUT), lambda i: (0, 0)),
        pl.BlockSpec((1, C_OUT), lambda i: (0, 0)),
    ],
    out_specs=[
        pl.BlockSpec((BLK_M, W2), lambda i: (i, 0)),
        pl.BlockSpec((BLK_M, C_OUT), lambda i: (i, 0)),
        pl.BlockSpec((BLK_M, 1), lambda i: (i, 0)),
    ],
    out_shape=[
        jax.ShapeDtypeStruct((N_NODES, W2), jnp.float32),
        jax.ShapeDtypeStruct((N_NODES, C_OUT), jnp.float32),
        jax.ShapeDtypeStruct((N_NODES, 1), jnp.float32),
    ],
)

_tc_fin = pl.pallas_call(
    _tc_fin_body,
    grid=(GRID_M,),
    in_specs=[
        pl.BlockSpec((BLK_M, C_OUT), lambda i: (i, 0)),
        pl.BlockSpec((NC, BLK_M, C_OUT), lambda i: (0, i, 0)),
        pl.BlockSpec((BLK_M, 1), lambda i: (i, 0)),
    ],
    out_specs=pl.BlockSpec((BLK_M, C_OUT), lambda i: (i, 0)),
    out_shape=jax.ShapeDtypeStruct((N_NODES, C_OUT), jnp.float32),
)


@jax.jit
def kernel(x, adj, W1_self, W1_neigh, b1, W2_self, W2_neigh, b2):
    src = adj[0]
    dst = adj[1]
    pad = E_PAD - N_EDGES
    src_p = jnp.concatenate([src, jnp.zeros((pad,), jnp.int32)])
    dst_p = jnp.concatenate([dst, jnp.full((pad,), N_NODES, jnp.int32)])
    src_p = src_p.reshape(NW, CH_PER_TILE, CHUNK)
    dst_p = dst_p.reshape(NW, CH_PER_TILE, CHUNK)

    table1 = jnp.concatenate(
        [x, jnp.ones((N_NODES, 1), jnp.float32),
         jnp.zeros((N_NODES, W1 - F_IN - 1), jnp.float32)], axis=1)

    z1 = jnp.zeros((ROWS_OUT, W1), jnp.float32)
    z2 = jnp.zeros((ROWS_OUT, W2), jnp.float32)

    parts1 = _sc_pass1(table1, src_p, dst_p, z1)
    hn, hs, dinv = _tc_mid(x, parts1, W1_self, W1_neigh, b1.reshape(1, HID),
                           W2_neigh, W2_self, b2.reshape(1, C_OUT))
    parts2 = _sc_pass2(hn, src_p, dst_p, z2)
    return _tc_fin(hs, parts2, dinv)


# speedup vs baseline: 8.8703x; 2.1897x over previous
import functools

import jax
import jax.numpy as jnp
from jax import lax
from jax.experimental import pallas as pl
from jax.experimental.pallas import tpu as pltpu
from jax.experimental.pallas import tpu_sc as plsc

N_NODES = 10000
N_EDGES = 320000
F_IN = 128
HID = 128
C_OUT = 64

NC = 2
NS = 16
NW = NC * NS

CHUNK = 128
EDGES_PER_TILE = 10240
CH_PER_TILE = EDGES_PER_TILE // CHUNK
E_PAD = EDGES_PER_TILE * NW
N_ACC = 10240
ROWS_OUT = N_ACC // NS
W1 = 144
W2 = 64

BLK_M = 1024
GRID_M = 10


def _make_sc_pass(width):
    mesh = plsc.VectorSubcoreMesh(core_axis_name="c", subcore_axis_name="s")

    @functools.partial(
        pl.kernel,
        out_type=jax.ShapeDtypeStruct((NC, N_ACC, width), jnp.float32),
        mesh=mesh,
        compiler_params=pltpu.CompilerParams(use_tc_tiling_on_sc=False),
        scratch_types=[
            pltpu.VMEM((CH_PER_TILE, CHUNK), jnp.int32),
            pltpu.VMEM((CH_PER_TILE, CHUNK), jnp.int32),
            pltpu.VMEM((CHUNK, width), jnp.float32),
            pltpu.VMEM_SHARED((N_ACC, width), jnp.float32),
        ],
    )
    def sc_pass(table, src_idx, dst_idx, zeros, out, src_v, dst_v, rows_v, acc):
        c = lax.axis_index("c")
        s = lax.axis_index("s")
        wid = c * NS + s
        pltpu.sync_copy(zeros, acc.at[pl.ds(s * ROWS_OUT, ROWS_OUT)])
        pltpu.sync_copy(src_idx.at[wid], src_v)
        pltpu.sync_copy(dst_idx.at[wid], dst_v)
        plsc.subcore_barrier()

        @pl.loop(0, CH_PER_TILE)
        def _(j):
            pltpu.sync_copy(table.at[src_v.at[j]], rows_v)
            pltpu.sync_copy(rows_v, acc.at[dst_v.at[j]], add=True)

        plsc.subcore_barrier()
        pltpu.sync_copy(
            acc.at[pl.ds(s * ROWS_OUT, ROWS_OUT)],
            out.at[c, pl.ds(s * ROWS_OUT, ROWS_OUT)],
        )

    return sc_pass


_sc_pass1 = _make_sc_pass(W1)
_sc_pass2 = _make_sc_pass(W2)


def _tc_mid_body(x_ref, parts_ref, w1s_ref, w1n_ref, b1_ref, w2n_ref, w2s_ref,
                 b2_ref, hn_ref, hs_ref, dinv_ref):
    p = parts_ref[0, :, :F_IN] + parts_ref[1, :, :F_IN]
    deg = parts_ref[0, :, F_IN:F_IN + 1] + parts_ref[1, :, F_IN:F_IN + 1]
    dinv = 1.0 / jnp.maximum(deg, 1.0)
    mean = p * dinv
    h = x_ref[...] @ w1s_ref[...] + mean @ w1n_ref[...] + b1_ref[...]
    h = jnp.maximum(h, 0.0)
    hn_ref[...] = h @ w2n_ref[...]
    hs_ref[...] = h @ w2s_ref[...] + b2_ref[...]
    dinv_ref[...] = dinv


def _tc_fin_body(hs_ref, parts_ref, dinv_ref, out_ref):
    agg = parts_ref[0] + parts_ref[1]
    out_ref[...] = hs_ref[...] + agg * dinv_ref[...]


_tc_mid = pl.pallas_call(
    _tc_mid_body,
    grid=(GRID_M,),
    in_specs=[
        pl.BlockSpec((BLK_M, F_IN), lambda i: (i, 0)),
        pl.BlockSpec((NC, BLK_M, W1), lambda i: (0, i, 0)),
        pl.BlockSpec((F_IN, HID), lambda i: (0, 0)),
        pl.BlockSpec((F_IN, HID), lambda i: (0, 0)),
        pl.BlockSpec((1, HID), lambda i: (0, 0)),
        pl.BlockSpec((HID, C_OUT), lambda i: (0, 0)),
        pl.BlockSpec((HID, C_OUT), lambda i: (0, 0)),
        pl.BlockSpec((1, C_OUT), lambda i: (0, 0)),
    ],
    out_specs=[
        pl.BlockSpec((BLK_M, W2), lambda i: (i, 0)),
        pl.BlockSpec((BLK_M, C_OUT), lambda i: (i, 0)),
        pl.BlockSpec((BLK_M, 1), lambda i: (i, 0)),
    ],
    out_shape=[
        jax.ShapeDtypeStruct((N_NODES, W2), jnp.float32),
        jax.ShapeDtypeStruct((N_NODES, C_OUT), jnp.float32),
        jax.ShapeDtypeStruct((N_NODES, 1), jnp.float32),
    ],
)

_tc_fin = pl.pallas_call(
    _tc_fin_body,
    grid=(GRID_M,),
    in_specs=[
        pl.BlockSpec((BLK_M, C_OUT), lambda i: (i, 0)),
        pl.BlockSpec((NC, BLK_M, C_OUT), lambda i: (0, i, 0)),
        pl.BlockSpec((BLK_M, 1), lambda i: (i, 0)),
    ],
    out_specs=pl.BlockSpec((BLK_M, C_OUT), lambda i: (i, 0)),
    out_shape=jax.ShapeDtypeStruct((N_NODES, C_OUT), jnp.float32),
)


@jax.jit
def kernel(x, adj, W1_self, W1_neigh, b1, W2_self, W2_neigh, b2):
    src = adj[0]
    dst = adj[1]
    pad = E_PAD - N_EDGES
    pad_i = jnp.arange(pad, dtype=jnp.int32)
    src_p = jnp.concatenate([src, pad_i % N_NODES])
    dst_p = jnp.concatenate([dst, N_NODES + pad_i % (N_ACC - N_NODES)])
    src_p = src_p.reshape(NW, CH_PER_TILE, CHUNK)
    dst_p = dst_p.reshape(NW, CH_PER_TILE, CHUNK)

    table1 = jnp.concatenate(
        [x, jnp.ones((N_NODES, 1), jnp.float32),
         jnp.zeros((N_NODES, W1 - F_IN - 1), jnp.float32)], axis=1)

    z1 = jnp.zeros((ROWS_OUT, W1), jnp.float32)
    z2 = jnp.zeros((ROWS_OUT, W2), jnp.float32)

    parts1 = _sc_pass1(table1, src_p, dst_p, z1)
    hn, hs, dinv = _tc_mid(x, parts1, W1_self, W1_neigh, b1.reshape(1, HID),
                           W2_neigh, W2_self, b2.reshape(1, C_OUT))
    parts2 = _sc_pass2(hn, src_p, dst_p, z2)
    return _tc_fin(hs, parts2, dinv)


# speedup vs baseline: 10.8301x; 1.2209x over previous
import functools

import jax
import jax.numpy as jnp
from jax import lax
from jax.experimental import pallas as pl
from jax.experimental.pallas import tpu as pltpu
from jax.experimental.pallas import tpu_sc as plsc

N_NODES = 10000
N_EDGES = 320000
F_IN = 128
HID = 128
C_OUT = 64

NC = 2
NS = 16
NW = NC * NS

CHUNK = 64
EDGES_PER_TILE = 10240
CH_PER_TILE = EDGES_PER_TILE // CHUNK
E_PAD = EDGES_PER_TILE * NW
N_ACC = 10240
ROWS_OUT = N_ACC // NS
W1 = 144
W2 = 64

NBUF = 2
BLK_M = 1024
GRID_M = 10


def _make_sc_pass(width):
    mesh = plsc.VectorSubcoreMesh(core_axis_name="c", subcore_axis_name="s")

    @functools.partial(
        pl.kernel,
        out_type=jax.ShapeDtypeStruct((NC, N_ACC, width), jnp.float32),
        mesh=mesh,
        compiler_params=pltpu.CompilerParams(use_tc_tiling_on_sc=False),
        scratch_types=[
            pltpu.VMEM((CH_PER_TILE, CHUNK), jnp.int32),
            pltpu.VMEM((CH_PER_TILE, CHUNK), jnp.int32),
            pltpu.VMEM((CHUNK, width), jnp.float32),
            pltpu.VMEM((CHUNK, width), jnp.float32),
            pltpu.VMEM_SHARED((N_ACC, width), jnp.float32),
        ] + [pltpu.SemaphoreType.DMA] * (2 * NBUF),
    )
    def sc_pass(table, src_idx, dst_idx, zeros, out, src_v, dst_v,
                rows0, rows1, acc, *sems):
        rows = (rows0, rows1)
        gs = sems[:NBUF]
        ss = sems[NBUF:]
        c = lax.axis_index("c")
        s = lax.axis_index("s")
        wid = c * NS + s
        pltpu.sync_copy(zeros, acc.at[pl.ds(s * ROWS_OUT, ROWS_OUT)])
        pltpu.sync_copy(src_idx.at[wid], src_v)
        pltpu.sync_copy(dst_idx.at[wid], dst_v)
        plsc.subcore_barrier()

        def gi(j, b):
            pltpu.async_copy(table.at[src_v.at[j]], rows[b], gs[b])

        def gw(b):
            pltpu.make_async_copy(table.at[src_v.at[0]], rows[b],
                                  gs[b]).wait()

        def si(j, b):
            pltpu.async_copy(rows[b], acc.at[dst_v.at[j]], ss[b],
                             add=True)

        def si_sync(j, b):
            pltpu.sync_copy(rows[b], acc.at[dst_v.at[j]], add=True)

        def sw(b):
            pltpu.make_async_copy(rows[b], acc.at[dst_v.at[0]],
                                  ss[b]).wait()

        gi(0, 0)
        gi(1, 1)

        @pl.loop(0, CH_PER_TILE // 2 - 1)
        def _(i):
            j0 = i * 2
            for b in range(2):
                j = j0 + b
                gw(b)
                si(j, b)
                sw(b)
                gi(j + 2, b)

        e = CH_PER_TILE - 2
        gw(0); si(e, 0); sw(0)
        gw(1); si(e + 1, 1); sw(1)

        plsc.subcore_barrier()
        pltpu.sync_copy(
            acc.at[pl.ds(s * ROWS_OUT, ROWS_OUT)],
            out.at[c, pl.ds(s * ROWS_OUT, ROWS_OUT)],
        )

    return sc_pass


_sc_pass1 = _make_sc_pass(W1)
_sc_pass2 = _make_sc_pass(W2)


def _tc_mid_body(x_ref, parts_ref, w1s_ref, w1n_ref, b1_ref, w2n_ref, w2s_ref,
                 b2_ref, hn_ref, hs_ref, dinv_ref):
    p = parts_ref[0, :, :F_IN] + parts_ref[1, :, :F_IN]
    deg = parts_ref[0, :, F_IN:F_IN + 1] + parts_ref[1, :, F_IN:F_IN + 1]
    dinv = 1.0 / jnp.maximum(deg, 1.0)
    mean = p * dinv
    h = x_ref[...] @ w1s_ref[...] + mean @ w1n_ref[...] + b1_ref[...]
    h = jnp.maximum(h, 0.0)
    hn_ref[...] = h @ w2n_ref[...]
    hs_ref[...] = h @ w2s_ref[...] + b2_ref[...]
    dinv_ref[...] = dinv


def _tc_fin_body(hs_ref, parts_ref, dinv_ref, out_ref):
    agg = parts_ref[0] + parts_ref[1]
    out_ref[...] = hs_ref[...] + agg * dinv_ref[...]


_tc_mid = pl.pallas_call(
    _tc_mid_body,
    grid=(GRID_M,),
    in_specs=[
        pl.BlockSpec((BLK_M, F_IN), lambda i: (i, 0)),
        pl.BlockSpec((NC, BLK_M, W1), lambda i: (0, i, 0)),
        pl.BlockSpec((F_IN, HID), lambda i: (0, 0)),
        pl.BlockSpec((F_IN, HID), lambda i: (0, 0)),
        pl.BlockSpec((1, HID), lambda i: (0, 0)),
        pl.BlockSpec((HID, C_OUT), lambda i: (0, 0)),
        pl.BlockSpec((HID, C_OUT), lambda i: (0, 0)),
        pl.BlockSpec((1, C_OUT), lambda i: (0, 0)),
    ],
    out_specs=[
        pl.BlockSpec((BLK_M, W2), lambda i: (i, 0)),
        pl.BlockSpec((BLK_M, C_OUT), lambda i: (i, 0)),
        pl.BlockSpec((BLK_M, 1), lambda i: (i, 0)),
    ],
    out_shape=[
        jax.ShapeDtypeStruct((N_NODES, W2), jnp.float32),
        jax.ShapeDtypeStruct((N_NODES, C_OUT), jnp.float32),
        jax.ShapeDtypeStruct((N_NODES, 1), jnp.float32),
    ],
)

_tc_fin = pl.pallas_call(
    _tc_fin_body,
    grid=(GRID_M,),
    in_specs=[
        pl.BlockSpec((BLK_M, C_OUT), lambda i: (i, 0)),
        pl.BlockSpec((NC, BLK_M, C_OUT), lambda i: (0, i, 0)),
        pl.BlockSpec((BLK_M, 1), lambda i: (i, 0)),
    ],
    out_specs=pl.BlockSpec((BLK_M, C_OUT), lambda i: (i, 0)),
    out_shape=jax.ShapeDtypeStruct((N_NODES, C_OUT), jnp.float32),
)


@jax.jit
def kernel(x, adj, W1_self, W1_neigh, b1, W2_self, W2_neigh, b2):
    src = adj[0]
    dst = adj[1]
    pad = E_PAD - N_EDGES
    pad_i = jnp.arange(pad, dtype=jnp.int32)
    src_p = jnp.concatenate([src, pad_i % N_NODES])
    dst_p = jnp.concatenate([dst, N_NODES + pad_i % (N_ACC - N_NODES)])
    src_p = src_p.reshape(NW, CH_PER_TILE, CHUNK)
    dst_p = dst_p.reshape(NW, CH_PER_TILE, CHUNK)

    table1 = jnp.concatenate(
        [x, jnp.ones((N_NODES, 1), jnp.float32),
         jnp.zeros((N_NODES, W1 - F_IN - 1), jnp.float32)], axis=1)

    z1 = jnp.zeros((ROWS_OUT, W1), jnp.float32)
    z2 = jnp.zeros((ROWS_OUT, W2), jnp.float32)

    parts1 = _sc_pass1(table1, src_p, dst_p, z1)
    hn, hs, dinv = _tc_mid(x, parts1, W1_self, W1_neigh, b1.reshape(1, HID),
                           W2_neigh, W2_self, b2.reshape(1, C_OUT))
    parts2 = _sc_pass2(hn, src_p, dst_p, z2)
    return _tc_fin(hs, parts2, dinv)
